# baseline (speedup 1.0000x reference)
import functools

import jax
import jax.numpy as jnp
from jax import lax
from jax.experimental import pallas as pl
from jax.experimental.pallas import tpu as pltpu
from jax.experimental.pallas import tpu_sc as plsc

B = 16384
D = 32
NC = 2
NS = 16
NW = NC * NS
BPW = B // NW

def _mesh():
    return plsc.VectorSubcoreMesh(core_axis_name="c", subcore_axis_name="s")


def _sc_gather_body(uid_hbm, iid_hbm, ut_hbm, it_hbm, ue_hbm, ie_hbm,
                    uidx_v, iidx_v, urows_v, irows_v, usem, isem):
    wid = lax.axis_index("s") * NC + lax.axis_index("c")
    base = wid * BPW
    pltpu.sync_copy(uid_hbm.at[pl.ds(base, BPW)], uidx_v)
    ucopy = pltpu.async_copy(ut_hbm.at[uidx_v], urows_v, usem)
    pltpu.sync_copy(iid_hbm.at[pl.ds(base, BPW)], iidx_v)
    icopy = pltpu.async_copy(it_hbm.at[iidx_v], irows_v, isem)
    ucopy.wait()
    pltpu.sync_copy(urows_v, ue_hbm.at[pl.ds(base, BPW)])
    icopy.wait()
    pltpu.sync_copy(irows_v, ie_hbm.at[pl.ds(base, BPW)])


def _sc_gather(user_ids, item_ids, user_table, item_table):
    emb = jax.ShapeDtypeStruct((B, D), jnp.float32)
    k = pl.kernel(
        _sc_gather_body,
        out_type=(emb, emb),
        mesh=_mesh(),
        compiler_params=pltpu.CompilerParams(use_tc_tiling_on_sc=False),
        scratch_types=[
            pltpu.VMEM((BPW,), jnp.int32),
            pltpu.VMEM((BPW,), jnp.int32),
            pltpu.VMEM((BPW, D), jnp.float32),
            pltpu.VMEM((BPW, D), jnp.float32),
            pltpu.SemaphoreType.DMA,
            pltpu.SemaphoreType.DMA,
        ],
    )
    return k(user_ids, item_ids, user_table, item_table)


def _mlp_body(ue_ref, ie_ref, W1_ref, b1_ref, W2_ref, b2_ref, W3_ref, b3_ref,
              out_ref):
    u = ue_ref[...]
    it = ie_ref[...]
    h = (jnp.dot(u, W1_ref[0:D, :], preferred_element_type=jnp.float32)
         + jnp.dot(it, W1_ref[D:2 * D, :], preferred_element_type=jnp.float32)
         + b1_ref[...])
    h = jnp.maximum(h, 0.0)
    h = jnp.dot(h, W2_ref[...], preferred_element_type=jnp.float32) + b2_ref[...]
    h = jnp.maximum(h, 0.0)
    out_ref[...] = (jnp.dot(h, W3_ref[...], preferred_element_type=jnp.float32)
                    + b3_ref[...])


def _tc_mlp(ue, ie, W1, b1, W2, b2, W3, b3):
    BB = 2048
    grid = (B // BB,)
    return pl.pallas_call(
        _mlp_body,
        grid=grid,
        in_specs=[
            pl.BlockSpec((BB, D), lambda i: (i, 0)),
            pl.BlockSpec((BB, D), lambda i: (i, 0)),
            pl.BlockSpec((2 * D, 256), lambda i: (0, 0)),
            pl.BlockSpec((1, 256), lambda i: (0, 0)),
            pl.BlockSpec((256, 64), lambda i: (0, 0)),
            pl.BlockSpec((1, 64), lambda i: (0, 0)),
            pl.BlockSpec((64, 1), lambda i: (0, 0)),
            pl.BlockSpec((1, 1), lambda i: (0, 0)),
        ],
        out_specs=pl.BlockSpec((BB, 1), lambda i: (i, 0)),
        out_shape=jax.ShapeDtypeStruct((B, 1), jnp.float32),
    )(ue, ie, W1, b1, W2, b2, W3, b3)


def kernel(user_ids, item_ids, user_table, item_table, W1, b1, W2, b2, W3, b3):
    ue, ie = _sc_gather(user_ids.astype(jnp.int32), item_ids.astype(jnp.int32),
                        user_table, item_table)
    return _tc_mlp(ue, ie, W1, b1.reshape(1, 256), W2, b2.reshape(1, 64),
                   W3, b3.reshape(1, 1))


# speedup vs baseline: 1.5890x; 1.5890x over previous
import jax
import jax.numpy as jnp
from jax import lax
from jax.experimental import pallas as pl
from jax.experimental.pallas import tpu as pltpu
from jax.experimental.pallas import tpu_sc as plsc

B = 16384
D = 32
NC = 2
NS = 16
NW = NC * NS
BPW = B // NW

PCK = 8192
PCK_ROWS = PCK // 4


def _mesh():
    return plsc.VectorSubcoreMesh(core_axis_name="c", subcore_axis_name="s")


def _pack_body(in_ref, out_ref):
    x = in_ref[...]
    xt = jnp.swapaxes(x, 0, 1)
    parts = [lax.slice(xt, (c * PCK_ROWS, 0), ((c + 1) * PCK_ROWS, D))
             for c in range(4)]
    out_ref[...] = jnp.concatenate(parts, axis=1)


def _pack(tbl_t):
    v = tbl_t.shape[1]
    steps = (v + PCK - 1) // PCK
    return pl.pallas_call(
        _pack_body,
        grid=(steps,),
        in_specs=[pl.BlockSpec((D, PCK), lambda i: (0, i))],
        out_specs=pl.BlockSpec((PCK_ROWS, 128), lambda i: (i, 0)),
        out_shape=jax.ShapeDtypeStruct((steps * PCK_ROWS, 128), jnp.float32),
    )(tbl_t)


def _sc_gather_body(uid_hbm, iid_hbm, pu_hbm, pi_hbm, ue_hbm, ie_hbm,
                    uidx_v, iidx_v, q4_v, rows_v, sem):
    wid = lax.axis_index("s") * NC + lax.axis_index("c")
    base = wid * BPW
    pltpu.sync_copy(uid_hbm.at[pl.ds(base, BPW)], uidx_v)
    pltpu.sync_copy(iid_hbm.at[pl.ds(base, BPW)], iidx_v)

    @pl.loop(0, BPW, step=16)
    def _(i):
        sl = pl.ds(i, 16)
        v = uidx_v[sl]
        q4_v[sl] = jax.lax.shift_left(
            jax.lax.shift_right_logical(v, 13), 11) | (v & 2047)

    pltpu.async_copy(pu_hbm.at[q4_v], rows_v, sem).wait()
    pltpu.sync_copy(rows_v, ue_hbm.at[pl.ds(base, BPW)])

    @pl.loop(0, BPW, step=16)
    def _(i):
        sl = pl.ds(i, 16)
        v = iidx_v[sl]
        q4_v[sl] = jax.lax.shift_left(
            jax.lax.shift_right_logical(v, 13), 11) | (v & 2047)

    pltpu.async_copy(pi_hbm.at[q4_v], rows_v, sem).wait()
    pltpu.sync_copy(rows_v, ie_hbm.at[pl.ds(base, BPW)])


def _sc_gather(user_ids, item_ids, pu, pi):
    emb = jax.ShapeDtypeStruct((B, 128), jnp.float32)
    k = pl.kernel(
        _sc_gather_body,
        out_type=(emb, emb),
        mesh=_mesh(),
        scratch_types=[
            pltpu.VMEM((BPW,), jnp.int32),
            pltpu.VMEM((BPW,), jnp.int32),
            pltpu.VMEM((BPW,), jnp.int32),
            pltpu.VMEM((BPW, 128), jnp.float32),
            pltpu.SemaphoreType.DMA,
        ],
    )
    return k(user_ids, item_ids, pu, pi)


def _select_slot(rows, off):
    emb = jnp.zeros((rows.shape[0], D), jnp.float32)
    for c in range(4):
        emb = emb + jnp.where(off == c, rows[:, c * D:(c + 1) * D], 0.0)
    return emb


def _mlp_body(ue_ref, ie_ref, uo_ref, io_ref, W1_ref, b1_ref, W2_ref, b2_ref,
              W3_ref, b3_ref, out_ref):
    u = _select_slot(ue_ref[...], uo_ref[...])
    it = _select_slot(ie_ref[...], io_ref[...])
    h = (jnp.dot(u, W1_ref[0:D, :], preferred_element_type=jnp.float32)
         + jnp.dot(it, W1_ref[D:2 * D, :], preferred_element_type=jnp.float32)
         + b1_ref[...])
    h = jnp.maximum(h, 0.0)
    h = jnp.dot(h, W2_ref[...], preferred_element_type=jnp.float32) + b2_ref[...]
    h = jnp.maximum(h, 0.0)
    out_ref[...] = (jnp.dot(h, W3_ref[...], preferred_element_type=jnp.float32)
                    + b3_ref[...])


def _tc_mlp(ue, ie, uo, io, W1, b1, W2, b2, W3, b3):
    BB = 2048
    grid = (B // BB,)
    return pl.pallas_call(
        _mlp_body,
        grid=grid,
        in_specs=[
            pl.BlockSpec((BB, 128), lambda i: (i, 0)),
            pl.BlockSpec((BB, 128), lambda i: (i, 0)),
            pl.BlockSpec((BB, 1), lambda i: (i, 0)),
            pl.BlockSpec((BB, 1), lambda i: (i, 0)),
            pl.BlockSpec((2 * D, 256), lambda i: (0, 0)),
            pl.BlockSpec((1, 256), lambda i: (0, 0)),
            pl.BlockSpec((256, 64), lambda i: (0, 0)),
            pl.BlockSpec((1, 64), lambda i: (0, 0)),
            pl.BlockSpec((64, 1), lambda i: (0, 0)),
            pl.BlockSpec((1, 1), lambda i: (0, 0)),
        ],
        out_specs=pl.BlockSpec((BB, 1), lambda i: (i, 0)),
        out_shape=jax.ShapeDtypeStruct((B, 1), jnp.float32),
    )(ue, ie, uo, io, W1, b1, W2, b2, W3, b3)


def kernel(user_ids, item_ids, user_table, item_table, W1, b1, W2, b2, W3, b3):
    uids = user_ids.astype(jnp.int32)
    iids = item_ids.astype(jnp.int32)
    pu = _pack(user_table.T)
    pi = _pack(item_table.T)
    ue, ie = _sc_gather(uids, iids, pu, pi)
    uo = ((uids >> 11) & 3).reshape(B, 1)
    io = ((iids >> 11) & 3).reshape(B, 1)
    return _tc_mlp(ue, ie, uo, io, W1, b1.reshape(1, 256), W2,
                   b2.reshape(1, 64), W3, b3.reshape(1, 1))


# speedup vs baseline: 2.4014x; 1.5113x over previous
import jax
import jax.numpy as jnp
from jax import lax
from jax.experimental import pallas as pl
from jax.experimental.pallas import tpu as pltpu
from jax.experimental.pallas import tpu_sc as plsc

B = 16384
D = 32
NC = 2
NS = 16
NW = NC * NS
BPW = B // NW

PCK = 8192
PCK_ROWS = PCK // 4


def _mesh():
    return plsc.VectorSubcoreMesh(core_axis_name="c", subcore_axis_name="s")


def _pack_body(in_ref, out_ref):
    x = in_ref[...]
    xs = jnp.concatenate(
        [lax.slice(x, (0, c * PCK_ROWS), (D, (c + 1) * PCK_ROWS))
         for c in range(4)], axis=0)
    eye = jnp.eye(128, dtype=jnp.float32)
    out_ref[...] = lax.dot_general(xs, eye, (((0,), (0,)), ((), ())),
                                   preferred_element_type=jnp.float32)


def _pack(tbl_t):
    v = tbl_t.shape[1]
    steps = (v + PCK - 1) // PCK
    return pl.pallas_call(
        _pack_body,
        grid=(steps,),
        in_specs=[pl.BlockSpec((D, PCK), lambda i: (0, i))],
        out_specs=pl.BlockSpec((PCK_ROWS, 128), lambda i: (i, 0)),
        out_shape=jax.ShapeDtypeStruct((steps * PCK_ROWS, 128), jnp.float32),
        compiler_params=pltpu.CompilerParams(fuse_transposed_lhs_in_matmul=True),
    )(tbl_t)


def _sc_gather_body(uid_hbm, iid_hbm, pu_hbm, pi_hbm, ue_hbm, ie_hbm,
                    uidx_v, iidx_v, q4_v, rows_v, sem):
    wid = lax.axis_index("s") * NC + lax.axis_index("c")
    base = wid * BPW
    pltpu.sync_copy(uid_hbm.at[pl.ds(base, BPW)], uidx_v)
    pltpu.sync_copy(iid_hbm.at[pl.ds(base, BPW)], iidx_v)

    @pl.loop(0, BPW, step=16)
    def _(i):
        sl = pl.ds(i, 16)
        v = uidx_v[sl]
        q4_v[sl] = jax.lax.shift_left(
            jax.lax.shift_right_logical(v, 13), 11) | (v & 2047)

    pltpu.async_copy(pu_hbm.at[q4_v], rows_v, sem).wait()
    pltpu.sync_copy(rows_v, ue_hbm.at[pl.ds(base, BPW)])

    @pl.loop(0, BPW, step=16)
    def _(i):
        sl = pl.ds(i, 16)
        v = iidx_v[sl]
        q4_v[sl] = jax.lax.shift_left(
            jax.lax.shift_right_logical(v, 13), 11) | (v & 2047)

    pltpu.async_copy(pi_hbm.at[q4_v], rows_v, sem).wait()
    pltpu.sync_copy(rows_v, ie_hbm.at[pl.ds(base, BPW)])


def _sc_gather(user_ids, item_ids, pu, pi):
    emb = jax.ShapeDtypeStruct((B, 128), jnp.float32)
    k = pl.kernel(
        _sc_gather_body,
        out_type=(emb, emb),
        mesh=_mesh(),
        scratch_types=[
            pltpu.VMEM((BPW,), jnp.int32),
            pltpu.VMEM((BPW,), jnp.int32),
            pltpu.VMEM((BPW,), jnp.int32),
            pltpu.VMEM((BPW, 128), jnp.float32),
            pltpu.SemaphoreType.DMA,
        ],
    )
    return k(user_ids, item_ids, pu, pi)


def _select_slot(rows, off):
    emb = jnp.zeros((rows.shape[0], D), jnp.float32)
    for c in range(4):
        emb = emb + jnp.where(off == c, rows[:, c * D:(c + 1) * D], 0.0)
    return emb


def _mlp_body(ue_ref, ie_ref, uo_ref, io_ref, W1_ref, b1_ref, W2_ref, b2_ref,
              W3_ref, b3_ref, out_ref):
    u = _select_slot(ue_ref[...], uo_ref[...])
    it = _select_slot(ie_ref[...], io_ref[...])
    h = (jnp.dot(u, W1_ref[0:D, :], preferred_element_type=jnp.float32)
         + jnp.dot(it, W1_ref[D:2 * D, :], preferred_element_type=jnp.float32)
         + b1_ref[...])
    h = jnp.maximum(h, 0.0)
    h = jnp.dot(h, W2_ref[...], preferred_element_type=jnp.float32) + b2_ref[...]
    h = jnp.maximum(h, 0.0)
    out_ref[...] = (jnp.dot(h, W3_ref[...], preferred_element_type=jnp.float32)
                    + b3_ref[...])


def _tc_mlp(ue, ie, uo, io, W1, b1, W2, b2, W3, b3):
    BB = 2048
    grid = (B // BB,)
    return pl.pallas_call(
        _mlp_body,
        grid=grid,
        in_specs=[
            pl.BlockSpec((BB, 128), lambda i: (i, 0)),
            pl.BlockSpec((BB, 128), lambda i: (i, 0)),
            pl.BlockSpec((BB, 1), lambda i: (i, 0)),
            pl.BlockSpec((BB, 1), lambda i: (i, 0)),
            pl.BlockSpec((2 * D, 256), lambda i: (0, 0)),
            pl.BlockSpec((1, 256), lambda i: (0, 0)),
            pl.BlockSpec((256, 64), lambda i: (0, 0)),
            pl.BlockSpec((1, 64), lambda i: (0, 0)),
            pl.BlockSpec((64, 1), lambda i: (0, 0)),
            pl.BlockSpec((1, 1), lambda i: (0, 0)),
        ],
        out_specs=pl.BlockSpec((BB, 1), lambda i: (i, 0)),
        out_shape=jax.ShapeDtypeStruct((B, 1), jnp.float32),
    )(ue, ie, uo, io, W1, b1, W2, b2, W3, b3)


def kernel(user_ids, item_ids, user_table, item_table, W1, b1, W2, b2, W3, b3):
    uids = user_ids.astype(jnp.int32)
    iids = item_ids.astype(jnp.int32)
    pu = _pack(user_table.T)
    pi = _pack(item_table.T)
    ue, ie = _sc_gather(uids, iids, pu, pi)
    uo = ((uids >> 11) & 3).reshape(B, 1)
    io = ((iids >> 11) & 3).reshape(B, 1)
    return _tc_mlp(ue, ie, uo, io, W1, b1.reshape(1, 256), W2,
                   b2.reshape(1, 64), W3, b3.reshape(1, 1))


# speedup vs baseline: 2.9361x; 1.2227x over previous
import jax
import jax.numpy as jnp
from jax import lax
from jax.experimental import pallas as pl
from jax.experimental.pallas import tpu as pltpu
from jax.experimental.pallas import tpu_sc as plsc

B = 16384
D = 32
NC = 2
NS = 16
NW = NC * NS
BPW = B // NW

PCK = 16384
PCK_ROWS = PCK // 4
PSH = PCK.bit_length() - 1
RSH = PCK_ROWS.bit_length() - 1
RMASK = PCK_ROWS - 1


def _mesh():
    return plsc.VectorSubcoreMesh(core_axis_name="c", subcore_axis_name="s")


def _pack_body(in_ref, out_ref):
    x = in_ref[...]
    xs = jnp.concatenate(
        [lax.slice(x, (0, c * PCK_ROWS), (D, (c + 1) * PCK_ROWS))
         for c in range(4)], axis=0)
    eye = jnp.eye(128, dtype=jnp.float32)
    out_ref[...] = lax.dot_general(xs, eye, (((0,), (0,)), ((), ())),
                                   preferred_element_type=jnp.float32)


def _pack(tbl_t):
    v = tbl_t.shape[1]
    steps = (v + PCK - 1) // PCK
    return pl.pallas_call(
        _pack_body,
        grid=(steps,),
        in_specs=[pl.BlockSpec((D, PCK), lambda i: (0, i))],
        out_specs=pl.BlockSpec((PCK_ROWS, 128), lambda i: (i, 0)),
        out_shape=jax.ShapeDtypeStruct((steps * PCK_ROWS, 128), jnp.float32),
        compiler_params=pltpu.CompilerParams(fuse_transposed_lhs_in_matmul=True),
    )(tbl_t)


def _sc_gather_body(uid_hbm, iid_hbm, pu_hbm, pi_hbm, ue_hbm, ie_hbm,
                    uidx_v, iidx_v, q4_v, rows_v, sem):
    wid = lax.axis_index("s") * NC + lax.axis_index("c")
    base = wid * BPW
    pltpu.sync_copy(uid_hbm.at[pl.ds(base, BPW)], uidx_v)
    pltpu.sync_copy(iid_hbm.at[pl.ds(base, BPW)], iidx_v)

    @pl.loop(0, BPW, step=16)
    def _(i):
        sl = pl.ds(i, 16)
        v = uidx_v[sl]
        q4_v[sl] = jax.lax.shift_left(
            jax.lax.shift_right_logical(v, PSH), RSH) | (v & RMASK)

    pltpu.async_copy(pu_hbm.at[q4_v], rows_v, sem).wait()
    pltpu.sync_copy(rows_v, ue_hbm.at[pl.ds(base, BPW)])

    @pl.loop(0, BPW, step=16)
    def _(i):
        sl = pl.ds(i, 16)
        v = iidx_v[sl]
        q4_v[sl] = jax.lax.shift_left(
            jax.lax.shift_right_logical(v, PSH), RSH) | (v & RMASK)

    pltpu.async_copy(pi_hbm.at[q4_v], rows_v, sem).wait()
    pltpu.sync_copy(rows_v, ie_hbm.at[pl.ds(base, BPW)])


def _sc_gather(user_ids, item_ids, pu, pi):
    emb = jax.ShapeDtypeStruct((B, 128), jnp.float32)
    k = pl.kernel(
        _sc_gather_body,
        out_type=(emb, emb),
        mesh=_mesh(),
        scratch_types=[
            pltpu.VMEM((BPW,), jnp.int32),
            pltpu.VMEM((BPW,), jnp.int32),
            pltpu.VMEM((BPW,), jnp.int32),
            pltpu.VMEM((BPW, 128), jnp.float32),
            pltpu.SemaphoreType.DMA,
        ],
    )
    return k(user_ids, item_ids, pu, pi)


def _select_slot(rows, off):
    emb = jnp.zeros((rows.shape[0], D), jnp.float32)
    for c in range(4):
        emb = emb + jnp.where(off == c, rows[:, c * D:(c + 1) * D], 0.0)
    return emb


def _mlp_body(ue_ref, ie_ref, uo_ref, io_ref, W1_ref, b1_ref, W2_ref, b2_ref,
              W3_ref, b3_ref, out_ref):
    u = _select_slot(ue_ref[...], uo_ref[...]).astype(jnp.bfloat16)
    it = _select_slot(ie_ref[...], io_ref[...]).astype(jnp.bfloat16)
    h = (jnp.dot(u, W1_ref[0:D, :], preferred_element_type=jnp.float32)
         + jnp.dot(it, W1_ref[D:2 * D, :], preferred_element_type=jnp.float32)
         + b1_ref[...])
    h = jnp.maximum(h, 0.0).astype(jnp.bfloat16)
    h = jnp.dot(h, W2_ref[...], preferred_element_type=jnp.float32) + b2_ref[...]
    h = jnp.maximum(h, 0.0)
    out_ref[...] = (jnp.dot(h, W3_ref[...], preferred_element_type=jnp.float32)
                    + b3_ref[...])


def _tc_mlp(ue, ie, uo, io, W1, b1, W2, b2, W3, b3):
    BB = 2048
    grid = (B // BB,)
    return pl.pallas_call(
        _mlp_body,
        grid=grid,
        in_specs=[
            pl.BlockSpec((BB, 128), lambda i: (i, 0)),
            pl.BlockSpec((BB, 128), lambda i: (i, 0)),
            pl.BlockSpec((BB, 1), lambda i: (i, 0)),
            pl.BlockSpec((BB, 1), lambda i: (i, 0)),
            pl.BlockSpec((2 * D, 256), lambda i: (0, 0)),
            pl.BlockSpec((1, 256), lambda i: (0, 0)),
            pl.BlockSpec((256, 64), lambda i: (0, 0)),
            pl.BlockSpec((1, 64), lambda i: (0, 0)),
            pl.BlockSpec((64, 1), lambda i: (0, 0)),
            pl.BlockSpec((1, 1), lambda i: (0, 0)),
        ],
        out_specs=pl.BlockSpec((BB, 1), lambda i: (i, 0)),
        out_shape=jax.ShapeDtypeStruct((B, 1), jnp.float32),
    )(ue, ie, uo, io, W1, b1, W2, b2, W3, b3)


def kernel(user_ids, item_ids, user_table, item_table, W1, b1, W2, b2, W3, b3):
    uids = user_ids.astype(jnp.int32)
    iids = item_ids.astype(jnp.int32)
    pu = _pack(user_table.T)
    pi = _pack(item_table.T)
    ue, ie = _sc_gather(uids, iids, pu, pi)
    uo = ((uids >> RSH) & 3).reshape(B, 1)
    io = ((iids >> RSH) & 3).reshape(B, 1)
    return _tc_mlp(ue, ie, uo, io, W1.astype(jnp.bfloat16),
                   b1.reshape(1, 256), W2.astype(jnp.bfloat16),
                   b2.reshape(1, 64), W3, b3.reshape(1, 1))


# speedup vs baseline: 3.8006x; 1.2944x over previous
import jax
import jax.numpy as jnp
from jax import lax
from jax.experimental import pallas as pl
from jax.experimental.pallas import tpu as pltpu
from jax.experimental.pallas import tpu_sc as plsc

B = 16384
D = 32
NC = 2
NS = 16
NW = NC * NS
BPW = B // NW

PCK = 32768
PCK_ROWS = PCK // 4
PSH = PCK.bit_length() - 1
RSH = PCK_ROWS.bit_length() - 1
RMASK = PCK_ROWS - 1


def _mesh():
    return plsc.VectorSubcoreMesh(core_axis_name="c", subcore_axis_name="s")


def _pack_body(in_ref, out_ref):
    x = in_ref[...]
    xs = jnp.concatenate(
        [lax.slice(x, (0, c * PCK_ROWS), (D, (c + 1) * PCK_ROWS))
         for c in range(4)], axis=0)
    eye = jnp.eye(128, dtype=jnp.float32)
    out_ref[...] = lax.dot_general(xs, eye, (((0,), (0,)), ((), ())),
                                   preferred_element_type=jnp.float32)


def _pack(tbl_t):
    v = tbl_t.shape[1]
    steps = (v + PCK - 1) // PCK
    return pl.pallas_call(
        _pack_body,
        grid=(steps,),
        in_specs=[pl.BlockSpec((D, PCK), lambda i: (0, i))],
        out_specs=pl.BlockSpec((PCK_ROWS, 128), lambda i: (i, 0)),
        out_shape=jax.ShapeDtypeStruct((steps * PCK_ROWS, 128), jnp.float32),
        compiler_params=pltpu.CompilerParams(fuse_transposed_lhs_in_matmul=True),
    )(tbl_t)


def _sc_gather_body(uid_hbm, iid_hbm, pu_hbm, pi_hbm, ue_hbm, ie_hbm,
                    uidx_v, iidx_v, q4_v, rows_v, sem):
    wid = lax.axis_index("s") * NC + lax.axis_index("c")
    base = wid * BPW
    pltpu.sync_copy(uid_hbm.at[pl.ds(base, BPW)], uidx_v)
    pltpu.sync_copy(iid_hbm.at[pl.ds(base, BPW)], iidx_v)

    @pl.loop(0, BPW, step=16)
    def _(i):
        sl = pl.ds(i, 16)
        v = uidx_v[sl]
        q4_v[sl] = jax.lax.shift_left(
            jax.lax.shift_right_logical(v, PSH), RSH) | (v & RMASK)

    pltpu.async_copy(pu_hbm.at[q4_v], rows_v, sem).wait()
    pltpu.sync_copy(rows_v, ue_hbm.at[pl.ds(base, BPW)])

    @pl.loop(0, BPW, step=16)
    def _(i):
        sl = pl.ds(i, 16)
        v = iidx_v[sl]
        q4_v[sl] = jax.lax.shift_left(
            jax.lax.shift_right_logical(v, PSH), RSH) | (v & RMASK)

    pltpu.async_copy(pi_hbm.at[q4_v], rows_v, sem).wait()
    pltpu.sync_copy(rows_v, ie_hbm.at[pl.ds(base, BPW)])


def _sc_gather(user_ids, item_ids, pu, pi):
    emb = jax.ShapeDtypeStruct((B, 128), jnp.float32)
    k = pl.kernel(
        _sc_gather_body,
        out_type=(emb, emb),
        mesh=_mesh(),
        scratch_types=[
            pltpu.VMEM((BPW,), jnp.int32),
            pltpu.VMEM((BPW,), jnp.int32),
            pltpu.VMEM((BPW,), jnp.int32),
            pltpu.VMEM((BPW, 128), jnp.float32),
            pltpu.SemaphoreType.DMA,
        ],
    )
    return k(user_ids, item_ids, pu, pi)


def _mlp_body(ue_ref, ie_ref, uo_ref, io_ref, W1u_ref, W1i_ref, b1_ref,
              W2_ref, b2_ref, W3_ref, b3_ref, out_ref):
    lane_slot = lax.shift_right_logical(
        lax.broadcasted_iota(jnp.int32, (1, 128), 1), 5)
    u = jnp.where(uo_ref[...] == lane_slot, ue_ref[...],
                  0.0).astype(jnp.bfloat16)
    it = jnp.where(io_ref[...] == lane_slot, ie_ref[...],
                   0.0).astype(jnp.bfloat16)
    h = (jnp.dot(u, W1u_ref[...], preferred_element_type=jnp.float32)
         + jnp.dot(it, W1i_ref[...], preferred_element_type=jnp.float32)
         + b1_ref[...])
    h = jnp.maximum(h, 0.0).astype(jnp.bfloat16)
    h = jnp.dot(h, W2_ref[...], preferred_element_type=jnp.float32) + b2_ref[...]
    h = jnp.maximum(h, 0.0)
    out_ref[...] = (lax.dot_general(W3_ref[...], h, (((0,), (1,)), ((), ())),
                                    preferred_element_type=jnp.float32)
                    + b3_ref[...])


def _tc_mlp(ue, ie, uo, io, W1u, W1i, b1, W2, b2, W3, b3):
    BB = 2048
    grid = (B // BB,)
    return pl.pallas_call(
        _mlp_body,
        grid=grid,
        in_specs=[
            pl.BlockSpec((BB, 128), lambda i: (i, 0)),
            pl.BlockSpec((BB, 128), lambda i: (i, 0)),
            pl.BlockSpec((BB, 1), lambda i: (i, 0)),
            pl.BlockSpec((BB, 1), lambda i: (i, 0)),
            pl.BlockSpec((128, 256), lambda i: (0, 0)),
            pl.BlockSpec((128, 256), lambda i: (0, 0)),
            pl.BlockSpec((1, 256), lambda i: (0, 0)),
            pl.BlockSpec((256, 64), lambda i: (0, 0)),
            pl.BlockSpec((1, 64), lambda i: (0, 0)),
            pl.BlockSpec((64, 1), lambda i: (0, 0)),
            pl.BlockSpec((1, 1), lambda i: (0, 0)),
        ],
        out_specs=pl.BlockSpec((1, BB), lambda i: (0, i)),
        out_shape=jax.ShapeDtypeStruct((1, B), jnp.float32),
    )(ue, ie, uo, io, W1u, W1i, b1, W2, b2, W3, b3)


def kernel(user_ids, item_ids, user_table, item_table, W1, b1, W2, b2, W3, b3):
    uids = user_ids.astype(jnp.int32)
    iids = item_ids.astype(jnp.int32)
    pu = _pack(user_table.T)
    pi = _pack(item_table.T)
    ue, ie = _sc_gather(uids, iids, pu, pi)
    uo = ((uids >> RSH) & 3).reshape(B, 1)
    io = ((iids >> RSH) & 3).reshape(B, 1)
    W1b = W1.astype(jnp.bfloat16)
    W1u = jnp.tile(W1b[0:D, :], (4, 1))
    W1i = jnp.tile(W1b[D:2 * D, :], (4, 1))
    out = _tc_mlp(ue, ie, uo, io, W1u, W1i, b1.reshape(1, 256),
                  W2.astype(jnp.bfloat16), b2.reshape(1, 64), W3,
                  b3.reshape(1, 1))
    return out.reshape(B, 1)


# speedup vs baseline: 3.8602x; 1.0157x over previous
import jax
import jax.numpy as jnp
from jax import lax
from jax.experimental import pallas as pl
from jax.experimental.pallas import tpu as pltpu
from jax.experimental.pallas import tpu_sc as plsc

B = 16384
D = 32
NC = 2
NS = 16
NW = NC * NS
BPW = B // NW

PCK = 32768
PCK_ROWS = PCK // 4
PSH = PCK.bit_length() - 1
RSH = PCK_ROWS.bit_length() - 1
RMASK = PCK_ROWS - 1


def _mesh():
    return plsc.VectorSubcoreMesh(core_axis_name="c", subcore_axis_name="s")


def _pack_body(in_ref, out_ref):
    x = in_ref[...]
    xs = jnp.concatenate(
        [lax.slice(x, (0, c * PCK_ROWS), (D, (c + 1) * PCK_ROWS))
         for c in range(4)], axis=0)
    eye = jnp.eye(128, dtype=jnp.float32)
    out_ref[...] = lax.dot_general(xs, eye, (((0,), (0,)), ((), ())),
                                   preferred_element_type=jnp.float32)


def _pack(tbl_t):
    v = tbl_t.shape[1]
    steps = (v + PCK - 1) // PCK
    return pl.pallas_call(
        _pack_body,
        grid=(steps,),
        in_specs=[pl.BlockSpec((D, PCK), lambda i: (0, i))],
        out_specs=pl.BlockSpec((PCK_ROWS, 128), lambda i: (i, 0)),
        out_shape=jax.ShapeDtypeStruct((steps * PCK_ROWS, 128), jnp.float32),
        compiler_params=pltpu.CompilerParams(fuse_transposed_lhs_in_matmul=True),
    )(tbl_t)


HCH = BPW // 2


def _sc_gather_body(ids_hbm, pt_hbm, emb_hbm, idx_v, q4_v, rows0_v, rows1_v,
                    sem0, sem1):
    wid = lax.axis_index("s") * NC + lax.axis_index("c")
    base = wid * BPW
    pltpu.sync_copy(ids_hbm.at[pl.ds(base, BPW)], idx_v)

    @pl.loop(0, BPW, step=16)
    def _(i):
        sl = pl.ds(i, 16)
        v = idx_v[sl]
        q4_v[sl] = jax.lax.shift_left(
            jax.lax.shift_right_logical(v, PSH), RSH) | (v & RMASK)

    c0 = pltpu.async_copy(pt_hbm.at[q4_v.at[pl.ds(0, HCH)]], rows0_v, sem0)
    c1 = pltpu.async_copy(pt_hbm.at[q4_v.at[pl.ds(HCH, HCH)]], rows1_v, sem1)
    c0.wait()
    pltpu.sync_copy(rows0_v, emb_hbm.at[pl.ds(base, HCH)])
    c1.wait()
    pltpu.sync_copy(rows1_v, emb_hbm.at[pl.ds(base + HCH, HCH)])


def _sc_gather(ids, pt):
    emb = jax.ShapeDtypeStruct((B, 128), jnp.float32)
    k = pl.kernel(
        _sc_gather_body,
        out_type=emb,
        mesh=_mesh(),
        scratch_types=[
            pltpu.VMEM((BPW,), jnp.int32),
            pltpu.VMEM((BPW,), jnp.int32),
            pltpu.VMEM((HCH, 128), jnp.float32),
            pltpu.VMEM((HCH, 128), jnp.float32),
            pltpu.SemaphoreType.DMA,
            pltpu.SemaphoreType.DMA,
        ],
    )
    return k(ids, pt)


def _mlp_body(ue_ref, ie_ref, uo_ref, io_ref, W1u_ref, W1i_ref, b1_ref,
              W2_ref, b2_ref, W3_ref, b3_ref, out_ref):
    lane_slot = lax.shift_right_logical(
        lax.broadcasted_iota(jnp.int32, (1, 128), 1), 5)
    u = jnp.where(uo_ref[...] == lane_slot, ue_ref[...],
                  0.0).astype(jnp.bfloat16)
    it = jnp.where(io_ref[...] == lane_slot, ie_ref[...],
                   0.0).astype(jnp.bfloat16)
    h = (jnp.dot(u, W1u_ref[...], preferred_element_type=jnp.float32)
         + jnp.dot(it, W1i_ref[...], preferred_element_type=jnp.float32)
         + b1_ref[...])
    h = jnp.maximum(h, 0.0).astype(jnp.bfloat16)
    h = jnp.dot(h, W2_ref[...], preferred_element_type=jnp.float32) + b2_ref[...]
    h = jnp.maximum(h, 0.0)
    out_ref[...] = (lax.dot_general(W3_ref[...], h, (((0,), (1,)), ((), ())),
                                    preferred_element_type=jnp.float32)
                    + b3_ref[...])


def _tc_mlp(ue, ie, uo, io, W1u, W1i, b1, W2, b2, W3, b3):
    BB = 2048
    grid = (B // BB,)
    return pl.pallas_call(
        _mlp_body,
        grid=grid,
        in_specs=[
            pl.BlockSpec((BB, 128), lambda i: (i, 0)),
            pl.BlockSpec((BB, 128), lambda i: (i, 0)),
            pl.BlockSpec((BB, 1), lambda i: (i, 0)),
            pl.BlockSpec((BB, 1), lambda i: (i, 0)),
            pl.BlockSpec((128, 256), lambda i: (0, 0)),
            pl.BlockSpec((128, 256), lambda i: (0, 0)),
            pl.BlockSpec((1, 256), lambda i: (0, 0)),
            pl.BlockSpec((256, 64), lambda i: (0, 0)),
            pl.BlockSpec((1, 64), lambda i: (0, 0)),
            pl.BlockSpec((64, 1), lambda i: (0, 0)),
            pl.BlockSpec((1, 1), lambda i: (0, 0)),
        ],
        out_specs=pl.BlockSpec((1, BB), lambda i: (0, i)),
        out_shape=jax.ShapeDtypeStruct((1, B), jnp.float32),
    )(ue, ie, uo, io, W1u, W1i, b1, W2, b2, W3, b3)


def kernel(user_ids, item_ids, user_table, item_table, W1, b1, W2, b2, W3, b3):
    uids = user_ids.astype(jnp.int32)
    iids = item_ids.astype(jnp.int32)
    pi = _pack(item_table.T)
    ie = _sc_gather(iids, pi)
    pu = _pack(user_table.T)
    ue = _sc_gather(uids, pu)
    uo = ((uids >> RSH) & 3).reshape(B, 1)
    io = ((iids >> RSH) & 3).reshape(B, 1)
    W1b = W1.astype(jnp.bfloat16)
    W1u = jnp.tile(W1b[0:D, :], (4, 1))
    W1i = jnp.tile(W1b[D:2 * D, :], (4, 1))
    out = _tc_mlp(ue, ie, uo, io, W1u, W1i, b1.reshape(1, 256),
                  W2.astype(jnp.bfloat16), b2.reshape(1, 64), W3,
                  b3.reshape(1, 1))
    return out.reshape(B, 1)


# speedup vs baseline: 4.3478x; 1.1263x over previous
import jax
import jax.numpy as jnp
from jax import lax
from jax.experimental import pallas as pl
from jax.experimental.pallas import tpu as pltpu
from jax.experimental.pallas import tpu_sc as plsc

B = 16384
D = 32
NC = 2
NS = 16
NW = NC * NS
BPW = B // NW

PCK = 32768
PCK_ROWS = PCK // 8
PSH = PCK.bit_length() - 1
RSH = PCK_ROWS.bit_length() - 1
RMASK = PCK_ROWS - 1


def _mesh():
    return plsc.VectorSubcoreMesh(core_axis_name="c", subcore_axis_name="s")


def _pack_body(in_ref, out_ref):
    x = in_ref[...]
    xh = lax.slice(x, (0, 0), (16, PCK))
    xl = lax.slice(x, (16, 0), (32, PCK))
    sh = jnp.concatenate(
        [lax.slice(xh, (0, e * PCK_ROWS), (16, (e + 1) * PCK_ROWS))
         for e in range(8)], axis=0).astype(jnp.bfloat16)
    sl = jnp.concatenate(
        [lax.slice(xl, (0, e * PCK_ROWS), (16, (e + 1) * PCK_ROWS))
         for e in range(8)], axis=0).astype(jnp.bfloat16)
    eye = jnp.eye(128, dtype=jnp.bfloat16)
    cdim = (((0,), (0,)), ((), ()))
    yh = lax.dot_general(sh, eye, cdim, preferred_element_type=jnp.float32)
    yl = lax.dot_general(sl, eye, cdim, preferred_element_type=jnp.float32)
    uh = lax.bitcast_convert_type(yh, jnp.uint32)
    ul = lax.bitcast_convert_type(yl, jnp.uint32)
    packed = (uh & jnp.uint32(0xFFFF0000)) | lax.shift_right_logical(
        ul, jnp.uint32(16))
    out_ref[...] = lax.bitcast_convert_type(packed, jnp.float32)


def _pack(tbl_t):
    v = tbl_t.shape[1]
    steps = (v + PCK - 1) // PCK
    return pl.pallas_call(
        _pack_body,
        grid=(steps,),
        in_specs=[pl.BlockSpec((D, PCK), lambda i: (0, i))],
        out_specs=pl.BlockSpec((PCK_ROWS, 128), lambda i: (i, 0)),
        out_shape=jax.ShapeDtypeStruct((steps * PCK_ROWS, 128), jnp.float32),
        compiler_params=pltpu.CompilerParams(fuse_transposed_lhs_in_matmul=True),
    )(tbl_t)


HCH = BPW // 2


def _sc_gather_body(ids_hbm, pt_hbm, emb_hbm, idx_v, q4_v, rows0_v, rows1_v,
                    sem0, sem1):
    wid = lax.axis_index("s") * NC + lax.axis_index("c")
    base = wid * BPW
    pltpu.sync_copy(ids_hbm.at[pl.ds(base, BPW)], idx_v)

    @pl.loop(0, BPW, step=16)
    def _(i):
        sl = pl.ds(i, 16)
        v = idx_v[sl]
        q4_v[sl] = jax.lax.shift_left(
            jax.lax.shift_right_logical(v, PSH), RSH) | (v & RMASK)

    c0 = pltpu.async_copy(pt_hbm.at[q4_v.at[pl.ds(0, HCH)]], rows0_v, sem0)
    c1 = pltpu.async_copy(pt_hbm.at[q4_v.at[pl.ds(HCH, HCH)]], rows1_v, sem1)
    c0.wait()
    pltpu.sync_copy(rows0_v, emb_hbm.at[pl.ds(base, HCH)])
    c1.wait()
    pltpu.sync_copy(rows1_v, emb_hbm.at[pl.ds(base + HCH, HCH)])


def _sc_gather(ids, pt):
    emb = jax.ShapeDtypeStruct((B, 128), jnp.float32)
    k = pl.kernel(
        _sc_gather_body,
        out_type=emb,
        mesh=_mesh(),
        scratch_types=[
            pltpu.VMEM((BPW,), jnp.int32),
            pltpu.VMEM((BPW,), jnp.int32),
            pltpu.VMEM((HCH, 128), jnp.float32),
            pltpu.VMEM((HCH, 128), jnp.float32),
            pltpu.SemaphoreType.DMA,
            pltpu.SemaphoreType.DMA,
        ],
    )
    return k(ids, pt)


def _unpack_masked(rows, off, lane_slot):
    sel = jnp.where(off == lane_slot, rows, 0.0)
    u32 = lax.bitcast_convert_type(sel, jnp.uint32)
    hi = lax.bitcast_convert_type(u32 & jnp.uint32(0xFFFF0000), jnp.float32)
    lo = lax.bitcast_convert_type(lax.shift_left(u32, jnp.uint32(16)),
                                  jnp.float32)
    return hi.astype(jnp.bfloat16), lo.astype(jnp.bfloat16)


def _mlp_body(ue_ref, ie_ref, uo_ref, io_ref, W1uh_ref, W1ul_ref, W1ih_ref,
              W1il_ref, b1_ref, W2_ref, b2_ref, W3_ref, b3_ref, out_ref):
    lane_slot = lax.shift_right_logical(
        lax.broadcasted_iota(jnp.int32, (1, 128), 1), 4)
    uh, ul = _unpack_masked(ue_ref[...], uo_ref[...], lane_slot)
    ih, il = _unpack_masked(ie_ref[...], io_ref[...], lane_slot)
    h = (jnp.dot(uh, W1uh_ref[...], preferred_element_type=jnp.float32)
         + jnp.dot(ul, W1ul_ref[...], preferred_element_type=jnp.float32)
         + jnp.dot(ih, W1ih_ref[...], preferred_element_type=jnp.float32)
         + jnp.dot(il, W1il_ref[...], preferred_element_type=jnp.float32)
         + b1_ref[...])
    h = jnp.maximum(h, 0.0).astype(jnp.bfloat16)
    h = jnp.dot(h, W2_ref[...], preferred_element_type=jnp.float32) + b2_ref[...]
    h = jnp.maximum(h, 0.0)
    out_ref[...] = (lax.dot_general(W3_ref[...], h, (((0,), (1,)), ((), ())),
                                    preferred_element_type=jnp.float32)
                    + b3_ref[...])


def _tc_mlp(ue, ie, uo, io, W1uh, W1ul, W1ih, W1il, b1, W2, b2, W3, b3):
    BB = 2048
    grid = (B // BB,)
    wspec = pl.BlockSpec((128, 256), lambda i: (0, 0))
    return pl.pallas_call(
        _mlp_body,
        grid=grid,
        in_specs=[
            pl.BlockSpec((BB, 128), lambda i: (i, 0)),
            pl.BlockSpec((BB, 128), lambda i: (i, 0)),
            pl.BlockSpec((BB, 1), lambda i: (i, 0)),
            pl.BlockSpec((BB, 1), lambda i: (i, 0)),
            wspec, wspec, wspec, wspec,
            pl.BlockSpec((1, 256), lambda i: (0, 0)),
            pl.BlockSpec((256, 64), lambda i: (0, 0)),
            pl.BlockSpec((1, 64), lambda i: (0, 0)),
            pl.BlockSpec((64, 1), lambda i: (0, 0)),
            pl.BlockSpec((1, 1), lambda i: (0, 0)),
        ],
        out_specs=pl.BlockSpec((1, BB), lambda i: (0, i)),
        out_shape=jax.ShapeDtypeStruct((1, B), jnp.float32),
    )(ue, ie, uo, io, W1uh, W1ul, W1ih, W1il, b1, W2, b2, W3, b3)


def kernel(user_ids, item_ids, user_table, item_table, W1, b1, W2, b2, W3, b3):
    uids = user_ids.astype(jnp.int32)
    iids = item_ids.astype(jnp.int32)
    pi = _pack(item_table.T)
    ie = _sc_gather(iids, pi)
    pu = _pack(user_table.T)
    ue = _sc_gather(uids, pu)
    uo = ((uids >> RSH) & 7).reshape(B, 1)
    io = ((iids >> RSH) & 7).reshape(B, 1)
    W1b = W1.astype(jnp.bfloat16)
    W1uh = jnp.tile(W1b[0:16, :], (8, 1))
    W1ul = jnp.tile(W1b[16:32, :], (8, 1))
    W1ih = jnp.tile(W1b[32:48, :], (8, 1))
    W1il = jnp.tile(W1b[48:64, :], (8, 1))
    out = _tc_mlp(ue, ie, uo, io, W1uh, W1ul, W1ih, W1il, b1.reshape(1, 256),
                  W2.astype(jnp.bfloat16), b2.reshape(1, 64), W3,
                  b3.reshape(1, 1))
    return out.reshape(B, 1)


# speedup vs baseline: 4.6520x; 1.0700x over previous
import dataclasses

import jax
import jax.numpy as jnp
from jax import lax
from jax.experimental import pallas as pl
from jax.experimental.pallas import tpu as pltpu
from jax.experimental.pallas import tpu_sc as plsc

B = 16384
D = 32
NC = 2
NS = 16
NW = NC * NS
BPW = B // NW

PCK = 32768
PCK_ROWS = PCK // 8
PSH = PCK.bit_length() - 1
RSH = PCK_ROWS.bit_length() - 1
RMASK = PCK_ROWS - 1


def _mesh():
    return plsc.VectorSubcoreMesh(core_axis_name="c", subcore_axis_name="s")


def _pack_body(in_ref, out_ref):
    x = in_ref[...]
    xh = lax.slice(x, (0, 0), (16, PCK))
    xl = lax.slice(x, (16, 0), (32, PCK))
    sh = jnp.concatenate(
        [lax.slice(xh, (0, e * PCK_ROWS), (16, (e + 1) * PCK_ROWS))
         for e in range(8)], axis=0).astype(jnp.bfloat16)
    sl = jnp.concatenate(
        [lax.slice(xl, (0, e * PCK_ROWS), (16, (e + 1) * PCK_ROWS))
         for e in range(8)], axis=0).astype(jnp.bfloat16)
    eye = jnp.eye(128, dtype=jnp.bfloat16)
    cdim = (((0,), (0,)), ((), ()))
    yh = lax.dot_general(sh, eye, cdim, preferred_element_type=jnp.float32)
    yl = lax.dot_general(sl, eye, cdim, preferred_element_type=jnp.float32)
    uh = lax.bitcast_convert_type(yh, jnp.uint32)
    ul = lax.bitcast_convert_type(yl, jnp.uint32)
    packed = (uh & jnp.uint32(0xFFFF0000)) | lax.shift_right_logical(
        ul, jnp.uint32(16))
    out_ref[...] = lax.bitcast_convert_type(packed, jnp.float32)


def _pack(tbl_t):
    v = tbl_t.shape[1]
    steps = (v + PCK - 1) // PCK
    return pl.pallas_call(
        _pack_body,
        grid=(steps,),
        in_specs=[pl.BlockSpec((D, PCK), lambda i: (0, i))],
        out_specs=pl.BlockSpec((PCK_ROWS, 128), lambda i: (i, 0)),
        out_shape=jax.ShapeDtypeStruct((steps * PCK_ROWS, 128), jnp.float32),
        compiler_params=pltpu.CompilerParams(fuse_transposed_lhs_in_matmul=True),
    )(tbl_t)


HCH = BPW // 2


def _extract_half(rows_v, idx_v, out_v, half_off):
    d_iota = lax.iota(jnp.int32, 16)

    @pl.loop(0, HCH, step=16)
    def _(j):
        v = idx_v[pl.ds(half_off + j, 16)]
        s = jax.lax.shift_left(
            jax.lax.shift_right_logical(v, RSH) & 7, 4)
        rows16 = j + d_iota
        for d in range(16):
            vals = plsc.load_gather(rows_v, [rows16, s + d])
            out_v[d, pl.ds(half_off + j, 16)] = vals


def _sc_gather_body(ids_hbm, pt_hbm, emb_hbm, idx_v, q4_v, rows0_v, rows1_v,
                    out_v, sem0, sem1):
    wid = lax.axis_index("s") * NC + lax.axis_index("c")
    base = wid * BPW
    pltpu.sync_copy(ids_hbm.at[pl.ds(base, BPW)], idx_v)

    @pl.loop(0, BPW, step=16)
    def _(i):
        sl = pl.ds(i, 16)
        v = idx_v[sl]
        q4_v[sl] = jax.lax.shift_left(
            jax.lax.shift_right_logical(v, PSH), RSH) | (v & RMASK)

    c0 = pltpu.async_copy(pt_hbm.at[q4_v.at[pl.ds(0, HCH)]], rows0_v, sem0)
    c1 = pltpu.async_copy(pt_hbm.at[q4_v.at[pl.ds(HCH, HCH)]], rows1_v, sem1)
    c0.wait()
    _extract_half(rows0_v, idx_v, out_v, 0)
    c1.wait()
    _extract_half(rows1_v, idx_v, out_v, HCH)
    pltpu.sync_copy(out_v, emb_hbm.at[:, pl.ds(base, BPW)])


def _sc_compiler_params():
    cp = pltpu.CompilerParams()
    if "needs_layout_passes" in pltpu.CompilerParams.__dataclass_fields__:
        cp = dataclasses.replace(cp, needs_layout_passes=False)
    return cp


def _sc_gather(ids, pt):
    emb = jax.ShapeDtypeStruct((16, B), jnp.float32)
    k = pl.kernel(
        _sc_gather_body,
        out_type=emb,
        mesh=_mesh(),
        compiler_params=_sc_compiler_params(),
        scratch_types=[
            pltpu.VMEM((BPW,), jnp.int32),
            pltpu.VMEM((BPW,), jnp.int32),
            pltpu.VMEM((HCH, 128), jnp.float32),
            pltpu.VMEM((HCH, 128), jnp.float32),
            pltpu.VMEM((16, BPW), jnp.float32),
            pltpu.SemaphoreType.DMA,
            pltpu.SemaphoreType.DMA,
        ],
    )
    return k(ids, pt)


def _unpack_t(x):
    u32 = lax.bitcast_convert_type(x, jnp.uint32)
    hi = lax.bitcast_convert_type(u32 & jnp.uint32(0xFFFF0000), jnp.float32)
    lo = lax.bitcast_convert_type(lax.shift_left(u32, jnp.uint32(16)),
                                  jnp.float32)
    return jnp.concatenate([hi, lo], axis=0).astype(jnp.bfloat16)


def _mlp_body(ue_ref, ie_ref, W1u_ref, W1i_ref, b1_ref, W2_ref, b2_ref,
              W3_ref, b3_ref, out_ref):
    cdim = (((0,), (0,)), ((), ()))
    u = _unpack_t(ue_ref[...])
    it = _unpack_t(ie_ref[...])
    h = (lax.dot_general(u, W1u_ref[...], cdim,
                         preferred_element_type=jnp.float32)
         + lax.dot_general(it, W1i_ref[...], cdim,
                           preferred_element_type=jnp.float32)
         + b1_ref[...])
    h = jnp.maximum(h, 0.0).astype(jnp.bfloat16)
    h = jnp.dot(h, W2_ref[...], preferred_element_type=jnp.float32) + b2_ref[...]
    h = jnp.maximum(h, 0.0)
    out_ref[...] = (lax.dot_general(W3_ref[...], h, (((0,), (1,)), ((), ())),
                                    preferred_element_type=jnp.float32)
                    + b3_ref[...])


def _tc_mlp(ue, ie, W1u, W1i, b1, W2, b2, W3, b3):
    BB = 2048
    grid = (B // BB,)
    return pl.pallas_call(
        _mlp_body,
        grid=grid,
        in_specs=[
            pl.BlockSpec((16, BB), lambda i: (0, i)),
            pl.BlockSpec((16, BB), lambda i: (0, i)),
            pl.BlockSpec((D, 256), lambda i: (0, 0)),
            pl.BlockSpec((D, 256), lambda i: (0, 0)),
            pl.BlockSpec((1, 256), lambda i: (0, 0)),
            pl.BlockSpec((256, 64), lambda i: (0, 0)),
            pl.BlockSpec((1, 64), lambda i: (0, 0)),
            pl.BlockSpec((64, 1), lambda i: (0, 0)),
            pl.BlockSpec((1, 1), lambda i: (0, 0)),
        ],
        out_specs=pl.BlockSpec((1, BB), lambda i: (0, i)),
        out_shape=jax.ShapeDtypeStruct((1, B), jnp.float32),
        compiler_params=pltpu.CompilerParams(fuse_transposed_lhs_in_matmul=True),
    )(ue, ie, W1u, W1i, b1, W2, b2, W3, b3)


def kernel(user_ids, item_ids, user_table, item_table, W1, b1, W2, b2, W3, b3):
    uids = user_ids.astype(jnp.int32)
    iids = item_ids.astype(jnp.int32)
    pi = _pack(item_table.T)
    ie = _sc_gather(iids, pi)
    pu = _pack(user_table.T)
    ue = _sc_gather(uids, pu)
    W1b = W1.astype(jnp.bfloat16)
    out = _tc_mlp(ue, ie, W1b[0:D, :], W1b[D:2 * D, :], b1.reshape(1, 256),
                  W2.astype(jnp.bfloat16), b2.reshape(1, 64), W3,
                  b3.reshape(1, 1))
    return out.reshape(B, 1)


# speedup vs baseline: 4.7009x; 1.0105x over previous
import dataclasses

import jax
import jax.numpy as jnp
from jax import lax
from jax.experimental import pallas as pl
from jax.experimental.pallas import tpu as pltpu
from jax.experimental.pallas import tpu_sc as plsc

B = 16384
D = 32
NC = 2
NS = 16
NW = NC * NS
BPW = B // NW

PCK = 32768
PCK_ROWS = PCK // 8
PSH = PCK.bit_length() - 1
RSH = PCK_ROWS.bit_length() - 1
RMASK = PCK_ROWS - 1


def _mesh():
    return plsc.VectorSubcoreMesh(core_axis_name="c", subcore_axis_name="s")


def _pack_body(in_ref, out_ref):
    x = in_ref[...]
    xh = lax.slice(x, (0, 0), (16, PCK))
    xl = lax.slice(x, (16, 0), (32, PCK))
    sh = jnp.concatenate(
        [lax.slice(xh, (0, e * PCK_ROWS), (16, (e + 1) * PCK_ROWS))
         for e in range(8)], axis=0).astype(jnp.bfloat16)
    sl = jnp.concatenate(
        [lax.slice(xl, (0, e * PCK_ROWS), (16, (e + 1) * PCK_ROWS))
         for e in range(8)], axis=0).astype(jnp.bfloat16)
    eye = jnp.eye(128, dtype=jnp.bfloat16)
    cdim = (((0,), (0,)), ((), ()))
    yh = lax.dot_general(sh, eye, cdim, preferred_element_type=jnp.float32)
    yl = lax.dot_general(sl, eye, cdim, preferred_element_type=jnp.float32)
    uh = lax.bitcast_convert_type(yh, jnp.uint32)
    ul = lax.bitcast_convert_type(yl, jnp.uint32)
    packed = (uh & jnp.uint32(0xFFFF0000)) | lax.shift_right_logical(
        ul, jnp.uint32(16))
    out_ref[...] = lax.bitcast_convert_type(packed, jnp.float32)


def _pack(tbl_t):
    v = tbl_t.shape[1]
    steps = (v + PCK - 1) // PCK
    return pl.pallas_call(
        _pack_body,
        grid=(steps,),
        in_specs=[pl.BlockSpec((D, PCK), lambda i: (0, i))],
        out_specs=pl.BlockSpec((PCK_ROWS, 128), lambda i: (i, 0)),
        out_shape=jax.ShapeDtypeStruct((steps * PCK_ROWS, 128), jnp.float32),
        compiler_params=pltpu.CompilerParams(fuse_transposed_lhs_in_matmul=True),
    )(tbl_t)


HCH = BPW // 2


def _extract_half(rows_v, idx_v, out_v, half_off):
    d_iota = lax.iota(jnp.int32, 16)

    @pl.loop(0, HCH, step=16)
    def _(j):
        v = idx_v[pl.ds(half_off + j, 16)]
        s = jax.lax.shift_left(
            jax.lax.shift_right_logical(v, RSH) & 7, 4)
        rows16 = j + d_iota
        for d in range(16):
            vals = plsc.load_gather(rows_v, [rows16, s + d])
            out_v[d, pl.ds(half_off + j, 16)] = vals


def _sc_gather_body(ids_hbm, pt_hbm, emb_hbm, idx_v, q4_v, rows0_v, rows1_v,
                    out_v, sem0, sem1):
    wid = lax.axis_index("s") * NC + lax.axis_index("c")
    base = wid * BPW
    pltpu.sync_copy(ids_hbm.at[pl.ds(base, BPW)], idx_v)

    @pl.loop(0, BPW, step=16)
    def _(i):
        sl = pl.ds(i, 16)
        v = idx_v[sl]
        q4_v[sl] = jax.lax.shift_left(
            jax.lax.shift_right_logical(v, PSH), RSH) | (v & RMASK)

    c0 = pltpu.async_copy(pt_hbm.at[q4_v.at[pl.ds(0, HCH)]], rows0_v, sem0)
    c1 = pltpu.async_copy(pt_hbm.at[q4_v.at[pl.ds(HCH, HCH)]], rows1_v, sem1)
    c0.wait()
    _extract_half(rows0_v, idx_v, out_v, 0)
    c1.wait()
    _extract_half(rows1_v, idx_v, out_v, HCH)
    pltpu.sync_copy(out_v, emb_hbm.at[:, pl.ds(base, BPW)])


def _sc_compiler_params():
    cp = pltpu.CompilerParams()
    if "needs_layout_passes" in pltpu.CompilerParams.__dataclass_fields__:
        cp = dataclasses.replace(cp, needs_layout_passes=False)
    return cp


def _sc_gather(ids, pt):
    emb = jax.ShapeDtypeStruct((16, B), jnp.float32)
    k = pl.kernel(
        _sc_gather_body,
        out_type=emb,
        mesh=_mesh(),
        compiler_params=_sc_compiler_params(),
        scratch_types=[
            pltpu.VMEM((BPW,), jnp.int32),
            pltpu.VMEM((BPW,), jnp.int32),
            pltpu.VMEM((HCH, 128), jnp.float32),
            pltpu.VMEM((HCH, 128), jnp.float32),
            pltpu.VMEM((16, BPW), jnp.float32),
            pltpu.SemaphoreType.DMA,
            pltpu.SemaphoreType.DMA,
        ],
    )
    return k(ids, pt)


def _unpack_t(x):
    u32 = lax.bitcast_convert_type(x, jnp.uint32)
    hi = lax.bitcast_convert_type(u32 & jnp.uint32(0xFFFF0000), jnp.float32)
    lo = lax.bitcast_convert_type(lax.shift_left(u32, jnp.uint32(16)),
                                  jnp.float32)
    return jnp.concatenate([hi, lo], axis=0).astype(jnp.bfloat16)


def _mlp_body(ue_ref, ie_ref, W1u_ref, W1i_ref, b1_ref, W2_ref, b2_ref,
              W3_ref, b3_ref, out_ref):
    cdim = (((0,), (0,)), ((), ()))
    u = _unpack_t(ue_ref[...])
    it = _unpack_t(ie_ref[...])
    h = (lax.dot_general(u, W1u_ref[...], cdim,
                         preferred_element_type=jnp.float32)
         + lax.dot_general(it, W1i_ref[...], cdim,
                           preferred_element_type=jnp.float32)
         + b1_ref[...])
    h = jnp.maximum(h, 0.0).astype(jnp.bfloat16)
    h = jnp.dot(h, W2_ref[...], preferred_element_type=jnp.float32) + b2_ref[...]
    h = jnp.maximum(h, 0.0)
    out_ref[...] = (lax.dot_general(W3_ref[...], h, (((0,), (1,)), ((), ())),
                                    preferred_element_type=jnp.float32)
                    + b3_ref[...])


def _tc_mlp(ue, ie, W1u, W1i, b1, W2, b2, W3, b3):
    BB = 4096
    grid = (B // BB,)
    return pl.pallas_call(
        _mlp_body,
        grid=grid,
        in_specs=[
            pl.BlockSpec((16, BB), lambda i: (0, i)),
            pl.BlockSpec((16, BB), lambda i: (0, i)),
            pl.BlockSpec((D, 256), lambda i: (0, 0)),
            pl.BlockSpec((D, 256), lambda i: (0, 0)),
            pl.BlockSpec((1, 256), lambda i: (0, 0)),
            pl.BlockSpec((256, 64), lambda i: (0, 0)),
            pl.BlockSpec((1, 64), lambda i: (0, 0)),
            pl.BlockSpec((64, 1), lambda i: (0, 0)),
            pl.BlockSpec((1, 1), lambda i: (0, 0)),
        ],
        out_specs=pl.BlockSpec((1, BB), lambda i: (0, i)),
        out_shape=jax.ShapeDtypeStruct((1, B), jnp.float32),
        compiler_params=pltpu.CompilerParams(fuse_transposed_lhs_in_matmul=True),
    )(ue, ie, W1u, W1i, b1, W2, b2, W3, b3)


def kernel(user_ids, item_ids, user_table, item_table, W1, b1, W2, b2, W3, b3):
    uids = user_ids.astype(jnp.int32)
    iids = item_ids.astype(jnp.int32)
    pi = _pack(item_table.T)
    ie = _sc_gather(iids, pi)
    pu = _pack(user_table.T)
    ue = _sc_gather(uids, pu)
    W1b = W1.astype(jnp.bfloat16)
    out = _tc_mlp(ue, ie, W1b[0:D, :], W1b[D:2 * D, :], b1.reshape(1, 256),
                  W2.astype(jnp.bfloat16), b2.reshape(1, 64), W3,
                  b3.reshape(1, 1))
    return out.reshape(B, 1)


# speedup vs baseline: 4.8332x; 1.0281x over previous
import dataclasses

import jax
import jax.numpy as jnp
from jax import lax
from jax.experimental import pallas as pl
from jax.experimental.pallas import tpu as pltpu
from jax.experimental.pallas import tpu_sc as plsc

B = 16384
D = 32
NC = 2
NS = 16
NW = NC * NS
BPW = B // NW

PCK = 65536
PCK_ROWS = PCK // 8
PSH = PCK.bit_length() - 1
RSH = PCK_ROWS.bit_length() - 1
RMASK = PCK_ROWS - 1


def _mesh():
    return plsc.VectorSubcoreMesh(core_axis_name="c", subcore_axis_name="s")


def _pack_body(in_ref, out_ref):
    x = in_ref[...]
    xh = lax.slice(x, (0, 0), (16, PCK))
    xl = lax.slice(x, (16, 0), (32, PCK))
    sh = jnp.concatenate(
        [lax.slice(xh, (0, e * PCK_ROWS), (16, (e + 1) * PCK_ROWS))
         for e in range(8)], axis=0).astype(jnp.bfloat16)
    sl = jnp.concatenate(
        [lax.slice(xl, (0, e * PCK_ROWS), (16, (e + 1) * PCK_ROWS))
         for e in range(8)], axis=0).astype(jnp.bfloat16)
    eye = jnp.eye(128, dtype=jnp.bfloat16)
    cdim = (((0,), (0,)), ((), ()))
    yh = lax.dot_general(sh, eye, cdim, preferred_element_type=jnp.float32)
    yl = lax.dot_general(sl, eye, cdim, preferred_element_type=jnp.float32)
    uh = lax.bitcast_convert_type(yh, jnp.uint32)
    ul = lax.bitcast_convert_type(yl, jnp.uint32)
    packed = (uh & jnp.uint32(0xFFFF0000)) | lax.shift_right_logical(
        ul, jnp.uint32(16))
    out_ref[...] = lax.bitcast_convert_type(packed, jnp.float32)


def _pack(tbl_t):
    v = tbl_t.shape[1]
    steps = (v + PCK - 1) // PCK
    return pl.pallas_call(
        _pack_body,
        grid=(steps,),
        in_specs=[pl.BlockSpec((D, PCK), lambda i: (0, i))],
        out_specs=pl.BlockSpec((PCK_ROWS, 128), lambda i: (i, 0)),
        out_shape=jax.ShapeDtypeStruct((steps * PCK_ROWS, 128), jnp.float32),
        compiler_params=pltpu.CompilerParams(fuse_transposed_lhs_in_matmul=True),
    )(tbl_t)


HCH = BPW // 2


def _extract_half(rows_v, idx_v, out_v, half_off):
    d_iota = lax.iota(jnp.int32, 16)

    @pl.loop(0, HCH, step=16)
    def _(j):
        v = idx_v[pl.ds(half_off + j, 16)]
        s = jax.lax.shift_left(
            jax.lax.shift_right_logical(v, RSH) & 7, 4)
        rows16 = j + d_iota
        for d in range(16):
            vals = plsc.load_gather(rows_v, [rows16, s + d])
            out_v[d, pl.ds(half_off + j, 16)] = vals


def _sc_gather_body(ids_hbm, pt_hbm, emb_hbm, idx_v, q4_v, rows0_v, rows1_v,
                    out_v, sem0, sem1):
    wid = lax.axis_index("s") * NC + lax.axis_index("c")
    base = wid * BPW
    pltpu.sync_copy(ids_hbm.at[pl.ds(base, BPW)], idx_v)

    @pl.loop(0, BPW, step=16)
    def _(i):
        sl = pl.ds(i, 16)
        v = idx_v[sl]
        q4_v[sl] = jax.lax.shift_left(
            jax.lax.shift_right_logical(v, PSH), RSH) | (v & RMASK)

    c0 = pltpu.async_copy(pt_hbm.at[q4_v.at[pl.ds(0, HCH)]], rows0_v, sem0)
    c1 = pltpu.async_copy(pt_hbm.at[q4_v.at[pl.ds(HCH, HCH)]], rows1_v, sem1)
    c0.wait()
    _extract_half(rows0_v, idx_v, out_v, 0)
    c1.wait()
    _extract_half(rows1_v, idx_v, out_v, HCH)
    pltpu.sync_copy(out_v, emb_hbm.at[:, pl.ds(base, BPW)])


def _sc_compiler_params():
    cp = pltpu.CompilerParams()
    if "needs_layout_passes" in pltpu.CompilerParams.__dataclass_fields__:
        cp = dataclasses.replace(cp, needs_layout_passes=False)
    return cp


def _sc_gather(ids, pt):
    emb = jax.ShapeDtypeStruct((16, B), jnp.float32)
    k = pl.kernel(
        _sc_gather_body,
        out_type=emb,
        mesh=_mesh(),
        compiler_params=_sc_compiler_params(),
        scratch_types=[
            pltpu.VMEM((BPW,), jnp.int32),
            pltpu.VMEM((BPW,), jnp.int32),
            pltpu.VMEM((HCH, 128), jnp.float32),
            pltpu.VMEM((HCH, 128), jnp.float32),
            pltpu.VMEM((16, BPW), jnp.float32),
            pltpu.SemaphoreType.DMA,
            pltpu.SemaphoreType.DMA,
        ],
    )
    return k(ids, pt)


def _unpack_t(x):
    u32 = lax.bitcast_convert_type(x, jnp.uint32)
    hi = lax.bitcast_convert_type(u32 & jnp.uint32(0xFFFF0000), jnp.float32)
    lo = lax.bitcast_convert_type(lax.shift_left(u32, jnp.uint32(16)),
                                  jnp.float32)
    return jnp.concatenate([hi, lo], axis=0).astype(jnp.bfloat16)


def _mlp_body(ue_ref, ie_ref, W1u_ref, W1i_ref, b1_ref, W2_ref, b2_ref,
              W3_ref, b3_ref, out_ref):
    cdim = (((0,), (0,)), ((), ()))
    u = _unpack_t(ue_ref[...])
    it = _unpack_t(ie_ref[...])
    h = (lax.dot_general(u, W1u_ref[...], cdim,
                         preferred_element_type=jnp.float32)
         + lax.dot_general(it, W1i_ref[...], cdim,
                           preferred_element_type=jnp.float32)
         + b1_ref[...])
    h = jnp.maximum(h, 0.0).astype(jnp.bfloat16)
    h = jnp.dot(h, W2_ref[...], preferred_element_type=jnp.float32) + b2_ref[...]
    h = jnp.maximum(h, 0.0)
    out_ref[...] = (lax.dot_general(W3_ref[...], h, (((0,), (1,)), ((), ())),
                                    preferred_element_type=jnp.float32)
                    + b3_ref[...])


def _tc_mlp(ue, ie, W1u, W1i, b1, W2, b2, W3, b3):
    BB = 4096
    grid = (B // BB,)
    return pl.pallas_call(
        _mlp_body,
        grid=grid,
        in_specs=[
            pl.BlockSpec((16, BB), lambda i: (0, i)),
            pl.BlockSpec((16, BB), lambda i: (0, i)),
            pl.BlockSpec((D, 256), lambda i: (0, 0)),
            pl.BlockSpec((D, 256), lambda i: (0, 0)),
            pl.BlockSpec((1, 256), lambda i: (0, 0)),
            pl.BlockSpec((256, 64), lambda i: (0, 0)),
            pl.BlockSpec((1, 64), lambda i: (0, 0)),
            pl.BlockSpec((64, 1), lambda i: (0, 0)),
            pl.BlockSpec((1, 1), lambda i: (0, 0)),
        ],
        out_specs=pl.BlockSpec((1, BB), lambda i: (0, i)),
        out_shape=jax.ShapeDtypeStruct((1, B), jnp.float32),
        compiler_params=pltpu.CompilerParams(fuse_transposed_lhs_in_matmul=True),
    )(ue, ie, W1u, W1i, b1, W2, b2, W3, b3)


def kernel(user_ids, item_ids, user_table, item_table, W1, b1, W2, b2, W3, b3):
    uids = user_ids.astype(jnp.int32)
    iids = item_ids.astype(jnp.int32)
    pi = _pack(item_table.T)
    ie = _sc_gather(iids, pi)
    pu = _pack(user_table.T)
    ue = _sc_gather(uids, pu)
    W1b = W1.astype(jnp.bfloat16)
    out = _tc_mlp(ue, ie, W1b[0:D, :], W1b[D:2 * D, :], b1.reshape(1, 256),
                  W2.astype(jnp.bfloat16), b2.reshape(1, 64), W3,
                  b3.reshape(1, 1))
    return out.reshape(B, 1)


# speedup vs baseline: 4.9774x; 1.0298x over previous
import dataclasses

import jax
import jax.numpy as jnp
from jax import lax
from jax.experimental import pallas as pl
from jax.experimental.pallas import tpu as pltpu
from jax.experimental.pallas import tpu_sc as plsc

B = 16384
D = 32
NC = 2
NS = 16
NW = NC * NS
BPW = B // NW

PCK = 131072
PCK_ROWS = PCK // 8
PSH = PCK.bit_length() - 1
RSH = PCK_ROWS.bit_length() - 1
RMASK = PCK_ROWS - 1


def _mesh():
    return plsc.VectorSubcoreMesh(core_axis_name="c", subcore_axis_name="s")


def _pack_body(in_ref, out_ref):
    x = in_ref[...]
    xh = lax.slice(x, (0, 0), (16, PCK))
    xl = lax.slice(x, (16, 0), (32, PCK))
    sh = jnp.concatenate(
        [lax.slice(xh, (0, e * PCK_ROWS), (16, (e + 1) * PCK_ROWS))
         for e in range(8)], axis=0).astype(jnp.bfloat16)
    sl = jnp.concatenate(
        [lax.slice(xl, (0, e * PCK_ROWS), (16, (e + 1) * PCK_ROWS))
         for e in range(8)], axis=0).astype(jnp.bfloat16)
    eye = jnp.eye(128, dtype=jnp.bfloat16)
    cdim = (((0,), (0,)), ((), ()))
    yh = lax.dot_general(sh, eye, cdim, preferred_element_type=jnp.float32)
    yl = lax.dot_general(sl, eye, cdim, preferred_element_type=jnp.float32)
    uh = lax.bitcast_convert_type(yh, jnp.uint32)
    ul = lax.bitcast_convert_type(yl, jnp.uint32)
    packed = (uh & jnp.uint32(0xFFFF0000)) | lax.shift_right_logical(
        ul, jnp.uint32(16))
    out_ref[...] = lax.bitcast_convert_type(packed, jnp.float32)


def _pack(tbl_t):
    v = tbl_t.shape[1]
    steps = (v + PCK - 1) // PCK
    return pl.pallas_call(
        _pack_body,
        grid=(steps,),
        in_specs=[pl.BlockSpec((D, PCK), lambda i: (0, i))],
        out_specs=pl.BlockSpec((PCK_ROWS, 128), lambda i: (i, 0)),
        out_shape=jax.ShapeDtypeStruct((steps * PCK_ROWS, 128), jnp.float32),
        compiler_params=pltpu.CompilerParams(fuse_transposed_lhs_in_matmul=True),
    )(tbl_t)


HCH = BPW // 2


def _extract_half(rows_v, idx_v, out_v, half_off):
    d_iota = lax.iota(jnp.int32, 16)

    @pl.loop(0, HCH, step=16)
    def _(j):
        v = idx_v[pl.ds(half_off + j, 16)]
        s = jax.lax.shift_left(
            jax.lax.shift_right_logical(v, RSH) & 7, 4)
        rows16 = j + d_iota
        for d in range(16):
            vals = plsc.load_gather(rows_v, [rows16, s + d])
            out_v[d, pl.ds(half_off + j, 16)] = vals


def _sc_gather_body(ids_hbm, pt_hbm, emb_hbm, idx_v, q4_v, rows0_v, rows1_v,
                    out_v, sem0, sem1):
    wid = lax.axis_index("s") * NC + lax.axis_index("c")
    base = wid * BPW
    pltpu.sync_copy(ids_hbm.at[pl.ds(base, BPW)], idx_v)

    @pl.loop(0, BPW, step=16)
    def _(i):
        sl = pl.ds(i, 16)
        v = idx_v[sl]
        q4_v[sl] = jax.lax.shift_left(
            jax.lax.shift_right_logical(v, PSH), RSH) | (v & RMASK)

    c0 = pltpu.async_copy(pt_hbm.at[q4_v.at[pl.ds(0, HCH)]], rows0_v, sem0)
    c1 = pltpu.async_copy(pt_hbm.at[q4_v.at[pl.ds(HCH, HCH)]], rows1_v, sem1)
    c0.wait()
    _extract_half(rows0_v, idx_v, out_v, 0)
    c1.wait()
    _extract_half(rows1_v, idx_v, out_v, HCH)
    pltpu.sync_copy(out_v, emb_hbm.at[:, pl.ds(base, BPW)])


def _sc_compiler_params():
    cp = pltpu.CompilerParams()
    if "needs_layout_passes" in pltpu.CompilerParams.__dataclass_fields__:
        cp = dataclasses.replace(cp, needs_layout_passes=False)
    return cp


def _sc_gather(ids, pt):
    emb = jax.ShapeDtypeStruct((16, B), jnp.float32)
    k = pl.kernel(
        _sc_gather_body,
        out_type=emb,
        mesh=_mesh(),
        compiler_params=_sc_compiler_params(),
        scratch_types=[
            pltpu.VMEM((BPW,), jnp.int32),
            pltpu.VMEM((BPW,), jnp.int32),
            pltpu.VMEM((HCH, 128), jnp.float32),
            pltpu.VMEM((HCH, 128), jnp.float32),
            pltpu.VMEM((16, BPW), jnp.float32),
            pltpu.SemaphoreType.DMA,
            pltpu.SemaphoreType.DMA,
        ],
    )
    return k(ids, pt)


def _unpack_t(x):
    u32 = lax.bitcast_convert_type(x, jnp.uint32)
    hi = lax.bitcast_convert_type(u32 & jnp.uint32(0xFFFF0000), jnp.float32)
    lo = lax.bitcast_convert_type(lax.shift_left(u32, jnp.uint32(16)),
                                  jnp.float32)
    return jnp.concatenate([hi, lo], axis=0).astype(jnp.bfloat16)


def _mlp_body(ue_ref, ie_ref, W1u_ref, W1i_ref, b1_ref, W2_ref, b2_ref,
              W3_ref, b3_ref, out_ref):
    cdim = (((0,), (0,)), ((), ()))
    u = _unpack_t(ue_ref[...])
    it = _unpack_t(ie_ref[...])
    h = (lax.dot_general(u, W1u_ref[...], cdim,
                         preferred_element_type=jnp.float32)
         + lax.dot_general(it, W1i_ref[...], cdim,
                           preferred_element_type=jnp.float32)
         + b1_ref[...])
    h = jnp.maximum(h, 0.0).astype(jnp.bfloat16)
    h = jnp.dot(h, W2_ref[...], preferred_element_type=jnp.float32) + b2_ref[...]
    h = jnp.maximum(h, 0.0)
    out_ref[...] = (lax.dot_general(W3_ref[...], h, (((0,), (1,)), ((), ())),
                                    preferred_element_type=jnp.float32)
                    + b3_ref[...])


def _tc_mlp(ue, ie, W1u, W1i, b1, W2, b2, W3, b3):
    BB = 4096
    grid = (B // BB,)
    return pl.pallas_call(
        _mlp_body,
        grid=grid,
        in_specs=[
            pl.BlockSpec((16, BB), lambda i: (0, i)),
            pl.BlockSpec((16, BB), lambda i: (0, i)),
            pl.BlockSpec((D, 256), lambda i: (0, 0)),
            pl.BlockSpec((D, 256), lambda i: (0, 0)),
            pl.BlockSpec((1, 256), lambda i: (0, 0)),
            pl.BlockSpec((256, 64), lambda i: (0, 0)),
            pl.BlockSpec((1, 64), lambda i: (0, 0)),
            pl.BlockSpec((64, 1), lambda i: (0, 0)),
            pl.BlockSpec((1, 1), lambda i: (0, 0)),
        ],
        out_specs=pl.BlockSpec((1, BB), lambda i: (0, i)),
        out_shape=jax.ShapeDtypeStruct((1, B), jnp.float32),
        compiler_params=pltpu.CompilerParams(fuse_transposed_lhs_in_matmul=True),
    )(ue, ie, W1u, W1i, b1, W2, b2, W3, b3)


def kernel(user_ids, item_ids, user_table, item_table, W1, b1, W2, b2, W3, b3):
    uids = user_ids.astype(jnp.int32)
    iids = item_ids.astype(jnp.int32)
    pi = _pack(item_table.T)
    ie = _sc_gather(iids, pi)
    pu = _pack(user_table.T)
    ue = _sc_gather(uids, pu)
    W1b = W1.astype(jnp.bfloat16)
    out = _tc_mlp(ue, ie, W1b[0:D, :], W1b[D:2 * D, :], b1.reshape(1, 256),
                  W2.astype(jnp.bfloat16), b2.reshape(1, 64), W3,
                  b3.reshape(1, 1))
    return out.reshape(B, 1)


# speedup vs baseline: 5.1460x; 1.0339x over previous
import dataclasses

import jax
import jax.numpy as jnp
from jax import lax
from jax.experimental import pallas as pl
from jax.experimental.pallas import tpu as pltpu
from jax.experimental.pallas import tpu_sc as plsc

B = 16384
D = 32
NC = 2
NS = 16
NW = NC * NS
BPW = B // NW

PCK = 131072
PCK_ROWS = PCK // 8
PSH = PCK.bit_length() - 1
RSH = PCK_ROWS.bit_length() - 1
RMASK = PCK_ROWS - 1


def _mesh():
    return plsc.VectorSubcoreMesh(core_axis_name="c", subcore_axis_name="s")


def _pack_body(in_ref, out_ref):
    x = in_ref[...]
    xh = lax.slice(x, (0, 0), (16, PCK))
    xl = lax.slice(x, (16, 0), (32, PCK))
    sh = jnp.concatenate(
        [lax.slice(xh, (0, e * PCK_ROWS), (16, (e + 1) * PCK_ROWS))
         for e in range(8)], axis=0).astype(jnp.bfloat16)
    sl = jnp.concatenate(
        [lax.slice(xl, (0, e * PCK_ROWS), (16, (e + 1) * PCK_ROWS))
         for e in range(8)], axis=0).astype(jnp.bfloat16)
    eye = jnp.eye(128, dtype=jnp.bfloat16)
    cdim = (((0,), (0,)), ((), ()))
    yh = lax.dot_general(sh, eye, cdim, preferred_element_type=jnp.float32)
    yl = lax.dot_general(sl, eye, cdim, preferred_element_type=jnp.float32)
    uh = lax.bitcast_convert_type(yh, jnp.uint32)
    ul = lax.bitcast_convert_type(yl, jnp.uint32)
    packed = (uh & jnp.uint32(0xFFFF0000)) | lax.shift_right_logical(
        ul, jnp.uint32(16))
    out_ref[...] = lax.bitcast_convert_type(packed, jnp.float32)


def _pack(tbl_t):
    v = tbl_t.shape[1]
    steps = (v + PCK - 1) // PCK
    return pl.pallas_call(
        _pack_body,
        grid=(steps,),
        in_specs=[pl.BlockSpec((D, PCK), lambda i: (0, i))],
        out_specs=pl.BlockSpec((PCK_ROWS, 128), lambda i: (i, 0)),
        out_shape=jax.ShapeDtypeStruct((steps * PCK_ROWS, 128), jnp.float32),
        compiler_params=pltpu.CompilerParams(fuse_transposed_lhs_in_matmul=True),
    )(tbl_t)


HCH = BPW // 2


def _extract_half(rows_v, idx_v, out_v, half_off):
    d_iota = lax.iota(jnp.int32, 16)

    @pl.loop(0, HCH, step=16)
    def _(j):
        v = idx_v[pl.ds(half_off + j, 16)]
        s = jax.lax.shift_left(
            jax.lax.shift_right_logical(v, RSH) & 7, 4)
        rows16 = j + d_iota
        for d in range(16):
            vals = plsc.load_gather(rows_v, [rows16, s + d])
            out_v[d, pl.ds(half_off + j, 16)] = vals


def _sc_gather_body(ids_hbm, pt_hbm, emb_hbm, idx_v, q4_v, rows0_v, rows1_v,
                    out_v, sem0, sem1):
    wid = lax.axis_index("s") * NC + lax.axis_index("c")
    base = wid * BPW
    pltpu.sync_copy(ids_hbm.at[pl.ds(base, BPW)], idx_v)

    @pl.loop(0, BPW, step=16)
    def _(i):
        sl = pl.ds(i, 16)
        v = idx_v[sl]
        q4_v[sl] = jax.lax.shift_left(
            jax.lax.shift_right_logical(v, PSH), RSH) | (v & RMASK)

    c0 = pltpu.async_copy(pt_hbm.at[q4_v.at[pl.ds(0, HCH)]], rows0_v, sem0)
    c1 = pltpu.async_copy(pt_hbm.at[q4_v.at[pl.ds(HCH, HCH)]], rows1_v, sem1)
    c0.wait()
    _extract_half(rows0_v, idx_v, out_v, 0)
    c1.wait()
    _extract_half(rows1_v, idx_v, out_v, HCH)
    pltpu.sync_copy(out_v, emb_hbm.at[:, pl.ds(base, BPW)])


def _sc_compiler_params():
    cp = pltpu.CompilerParams()
    if "needs_layout_passes" in pltpu.CompilerParams.__dataclass_fields__:
        cp = dataclasses.replace(cp, needs_layout_passes=False)
    return cp


def _sc_gather(ids, pt):
    emb = jax.ShapeDtypeStruct((16, B), jnp.float32)
    k = pl.kernel(
        _sc_gather_body,
        out_type=emb,
        mesh=_mesh(),
        compiler_params=_sc_compiler_params(),
        scratch_types=[
            pltpu.VMEM((BPW,), jnp.int32),
            pltpu.VMEM((BPW,), jnp.int32),
            pltpu.VMEM((HCH, 128), jnp.float32),
            pltpu.VMEM((HCH, 128), jnp.float32),
            pltpu.VMEM((16, BPW), jnp.float32),
            pltpu.SemaphoreType.DMA,
            pltpu.SemaphoreType.DMA,
        ],
    )
    return k(ids, pt)


def _unpack_t(x):
    u32 = lax.bitcast_convert_type(x, jnp.uint32)
    hi = lax.bitcast_convert_type(u32 & jnp.uint32(0xFFFF0000), jnp.float32)
    lo = lax.bitcast_convert_type(lax.shift_left(u32, jnp.uint32(16)),
                                  jnp.float32)
    return jnp.concatenate([hi, lo], axis=0).astype(jnp.bfloat16)


def _mlp_body(ue_ref, ie_ref, W1_ref, b1_ref, W2_ref, b2_ref,
              W3_ref, b3_ref, out_ref):
    cdim = (((0,), (0,)), ((), ()))
    u = _unpack_t(ue_ref[...])
    it = _unpack_t(ie_ref[...])
    ui = jnp.concatenate([u, it], axis=0)
    h = (lax.dot_general(ui, W1_ref[...], cdim,
                         preferred_element_type=jnp.float32)
         + b1_ref[...])
    h = jnp.maximum(h, 0.0).astype(jnp.bfloat16)
    h = jnp.dot(h, W2_ref[...], preferred_element_type=jnp.float32) + b2_ref[...]
    h = jnp.maximum(h, 0.0)
    out_ref[...] = (lax.dot_general(W3_ref[...], h, (((0,), (1,)), ((), ())),
                                    preferred_element_type=jnp.float32)
                    + b3_ref[...])


def _tc_mlp(ue, ie, W1, b1, W2, b2, W3, b3):
    BB = 4096
    grid = (B // BB,)
    return pl.pallas_call(
        _mlp_body,
        grid=grid,
        in_specs=[
            pl.BlockSpec((16, BB), lambda i: (0, i)),
            pl.BlockSpec((16, BB), lambda i: (0, i)),
            pl.BlockSpec((2 * D, 256), lambda i: (0, 0)),
            pl.BlockSpec((1, 256), lambda i: (0, 0)),
            pl.BlockSpec((256, 64), lambda i: (0, 0)),
            pl.BlockSpec((1, 64), lambda i: (0, 0)),
            pl.BlockSpec((64, 1), lambda i: (0, 0)),
            pl.BlockSpec((1, 1), lambda i: (0, 0)),
        ],
        out_specs=pl.BlockSpec((1, BB), lambda i: (0, i)),
        out_shape=jax.ShapeDtypeStruct((1, B), jnp.float32),
        compiler_params=pltpu.CompilerParams(fuse_transposed_lhs_in_matmul=True),
    )(ue, ie, W1, b1, W2, b2, W3, b3)


def kernel(user_ids, item_ids, user_table, item_table, W1, b1, W2, b2, W3, b3):
    uids = user_ids.astype(jnp.int32)
    iids = item_ids.astype(jnp.int32)
    pi = _pack(item_table.T)
    ie = _sc_gather(iids, pi)
    pu = _pack(user_table.T)
    ue = _sc_gather(uids, pu)
    out = _tc_mlp(ue, ie, W1.astype(jnp.bfloat16), b1.reshape(1, 256),
                  W2.astype(jnp.bfloat16), b2.reshape(1, 64), W3,
                  b3.reshape(1, 1))
    return out.reshape(B, 1)
